# baseline (speedup 1.0000x reference)
import jax
import jax.numpy as jnp
from jax import lax
from jax.experimental import pallas as pl
from jax.experimental.pallas import tpu as pltpu
from jax.experimental.pallas import tpu_sc as plsc

_B = 16384
_D = 64
_NC, _NS = 2, 16
_NW = _NC * _NS
_BPW = _B // _NW
_CH = 128
_NCH = _BPW // _CH

_BB = 1024
_FEAT = 128
_H = 128


def _gather_body(ublk_hbm, mblk_hbm, utab2_hbm, mtab2_hbm,
                 urows_hbm, mrows_hbm,
                 uidx_v, midx_v, rows_v, sem):
    wid = lax.axis_index("s") * _NC + lax.axis_index("c")
    base = wid * _BPW
    pltpu.sync_copy(ublk_hbm.at[pl.ds(wid * _NCH, _NCH)], uidx_v)
    pltpu.sync_copy(mblk_hbm.at[pl.ds(wid * _NCH, _NCH)], midx_v)
    for tab, idx_v, rows_hbm in (
        (utab2_hbm, uidx_v, urows_hbm),
        (mtab2_hbm, midx_v, mrows_hbm),
    ):
        copies = []
        for j in range(_NCH):
            copies.append(pltpu.async_copy(
                tab.at[idx_v.at[j]], rows_v.at[pl.ds(j * _CH, _CH)], sem))
        for c in copies:
            c.wait()
        pltpu.sync_copy(rows_v, rows_hbm.at[pl.ds(base, _BPW)])


@jax.jit
def _sc_gather(ublk2d, mblk2d, utab2, mtab2):
    mesh = plsc.VectorSubcoreMesh(core_axis_name="c", subcore_axis_name="s")
    return pl.kernel(
        _gather_body,
        mesh=mesh,
        out_type=[
            jax.ShapeDtypeStruct((_B, 128), jnp.float32),
            jax.ShapeDtypeStruct((_B, 128), jnp.float32),
        ],
        scratch_types=[
            pltpu.VMEM((_NCH, _CH), jnp.int32),
            pltpu.VMEM((_NCH, _CH), jnp.int32),
            pltpu.VMEM((_BPW, 128), jnp.float32),
            pltpu.SemaphoreType.DMA,
        ],
    )(ublk2d, mblk2d, utab2, mtab2)


def _mlp_body(uw_ref, mw_ref, upar_ref, mpar_ref, mf_ref, wf_ref,
              w1u_ref, w1m_ref, w1f_ref, b1_ref, w2_ref, b2_ref, out_ref):
    upar = upar_ref[...].reshape(_BB, 1)
    mpar = mpar_ref[...].reshape(_BB, 1)
    uv = jnp.where(upar == 0, uw_ref[:, :_D], uw_ref[:, _D:])
    mv = jnp.where(mpar == 0, mw_ref[:, :_D], mw_ref[:, _D:])
    fv = jnp.dot(mf_ref[...], wf_ref[...], preferred_element_type=jnp.float32)
    acc = jnp.dot(uv, w1u_ref[...], preferred_element_type=jnp.float32)
    acc = acc + jnp.dot(mv, w1m_ref[...], preferred_element_type=jnp.float32)
    acc = acc + jnp.dot(fv, w1f_ref[...], preferred_element_type=jnp.float32)
    acc = acc + b1_ref[...]
    h = jnp.maximum(acc, 0.0)
    out_ref[...] = jnp.sum(h * w2_ref[...], axis=1) + b2_ref[0, 0]


def _mlp(uw, mw, upar, mpar, mf, wf, w1u, w1m, w1f, b1p, w2row, b2):
    grid = (_B // _BB,)
    full = lambda i: (0, 0)
    return pl.pallas_call(
        _mlp_body,
        grid=grid,
        in_specs=[
            pl.BlockSpec((_BB, 128), lambda i: (i, 0)),
            pl.BlockSpec((_BB, 128), lambda i: (i, 0)),
            pl.BlockSpec((_BB,), lambda i: (i,)),
            pl.BlockSpec((_BB,), lambda i: (i,)),
            pl.BlockSpec((_BB, _FEAT), lambda i: (i, 0)),
            pl.BlockSpec((_FEAT, _D), full),
            pl.BlockSpec((_D, _H), full),
            pl.BlockSpec((_D, _H), full),
            pl.BlockSpec((_D, _H), full),
            pl.BlockSpec((1, _H), full),
            pl.BlockSpec((1, _H), full),
            pl.BlockSpec((1, 1), full),
        ],
        out_specs=pl.BlockSpec((_BB,), lambda i: (i,)),
        out_shape=jax.ShapeDtypeStruct((_B,), jnp.float32),
    )(uw, mw, upar, mpar, mf, wf, w1u, w1m, w1f, b1p, w2row, b2)


def kernel(user_ids, movie_ids, movie_features, user_table, movie_table,
           W_feat, b_feat, W1, b1, W2, b2):
    uids = user_ids.astype(jnp.int32)
    mids = movie_ids.astype(jnp.int32)
    utab2 = user_table.reshape(-1, 128)
    mtab2 = movie_table.reshape(-1, 128)
    ublk = (uids >> 1).reshape(_B // _CH, _CH)
    mblk = (mids >> 1).reshape(_B // _CH, _CH)
    uw, mw = _sc_gather(ublk, mblk, utab2, mtab2)
    w1u = W1[:_D]
    w1m = W1[_D:2 * _D]
    w1f = W1[2 * _D:]
    b1p = (b1 + b_feat @ w1f).reshape(1, _H)
    out = _mlp(uw, mw, uids & 1, mids & 1, movie_features, W_feat,
               w1u, w1m, w1f, b1p, W2.reshape(1, _H), b2.reshape(1, 1))
    return out
